# baseline (speedup 1.0000x reference)
import jax
import jax.numpy as jnp
from jax import lax
from jax.experimental import pallas as pl


def _wta_gate(v_col):
    n = v_col.shape[0]
    ratio = 0.9 if n == 16 else 0.8
    k = int(n * ratio)
    a = jnp.broadcast_to(v_col, (n, n))
    b = jnp.transpose(a)
    row = lax.broadcasted_iota(jnp.int32, (n, n), 0)
    col = lax.broadcasted_iota(jnp.int32, (n, n), 1)
    beats = (b > a) | ((b == a) & (col < row))
    rank = jnp.sum(beats.astype(jnp.float32), axis=1, keepdims=True)
    mask = jnp.where(rank < float(k), 1.0, 0.0)
    return v_col * mask


def _rm_sew_body(x_ref, wt1_ref, wt2_ref, wc1t_ref, wc2t_ref, o_ref):
    xb = x_ref[0]
    f, c, hw = xb.shape
    s = jnp.sum(xb, axis=-1)
    mx = jnp.max(xb, axis=-1)

    avg_t = jnp.sum(s, axis=1, keepdims=True) * (1.0 / (c * hw))
    max_t = jnp.max(mx, axis=1, keepdims=True)
    vt = jnp.concatenate([avg_t, max_t], axis=1)
    ht = jnp.maximum(jnp.dot(wt1_ref[...], vt,
                             preferred_element_type=jnp.float32), 0.0)
    ot = jnp.dot(wt2_ref[...], ht, preferred_element_type=jnp.float32)
    ta = jax.nn.sigmoid(ot[:, 0:1] + ot[:, 1:2])

    avg_c = jnp.sum(ta * s, axis=0, keepdims=True) * (1.0 / (f * hw))
    max_c = jnp.max(ta * mx, axis=0, keepdims=True)
    vc = jnp.concatenate([avg_c, max_c], axis=0)
    hc = jnp.maximum(jnp.dot(vc, wc1t_ref[...],
                             preferred_element_type=jnp.float32), 0.0)
    oc = jnp.dot(hc, wc2t_ref[...], preferred_element_type=jnp.float32)
    ca = jax.nn.sigmoid(oc[0:1, :] + oc[1:2, :])

    g_t = _wta_gate(ta)
    g_c = _wta_gate(jnp.transpose(ca))

    for i in range(f):
        scale = g_c * g_t[i:i + 1, 0:1]
        o_ref[0, i] = xb[i] * scale


def kernel(x, w_ta1, w_ta2, w_ca1, w_ca2):
    b, f, c, h, w = x.shape
    hw = h * w
    x4 = x.reshape(b, f, c, hw)
    out4 = pl.pallas_call(
        _rm_sew_body,
        grid=(b,),
        in_specs=[
            pl.BlockSpec((1, f, c, hw), lambda i: (i, 0, 0, 0)),
            pl.BlockSpec((f, f), lambda i: (0, 0)),
            pl.BlockSpec((f, f), lambda i: (0, 0)),
            pl.BlockSpec((c, c), lambda i: (0, 0)),
            pl.BlockSpec((c, c), lambda i: (0, 0)),
        ],
        out_specs=pl.BlockSpec((1, f, c, hw), lambda i: (i, 0, 0, 0)),
        out_shape=jax.ShapeDtypeStruct((b, f, c, hw), x.dtype),
    )(x4, w_ta1, w_ta2, w_ca1.T, w_ca2.T)
    return out4.reshape(b, f, c, h, w)
